# baseline (speedup 1.0000x reference)
import functools

import jax
import jax.numpy as jnp
from jax import lax
from jax.experimental import pallas as pl
from jax.experimental.pallas import tpu as pltpu
from jax.experimental.pallas import tpu_sc as plsc

HIDDEN = 2048
E = 16
TOPK = 2
DFF = 1408
SH = DFF * 2
S = 2048
PAIRS = S * TOPK
TR = 256
NTILES = PAIRS // TR
GSTEPS = NTILES + E - 1
SHB = 256
NEG = -1e30


def _router_body(x_ref, rwt_ref, a1_ref, a2_ref, w1_ref, w2_ref):
    logits = jnp.dot(x_ref[...], rwt_ref[...], preferred_element_type=jnp.float32)
    iota = lax.broadcasted_iota(jnp.int32, (S, E), 1)
    m1 = jnp.max(logits, axis=1, keepdims=True)
    a1 = jnp.min(jnp.where(logits == m1, iota, E), axis=1, keepdims=True)
    masked = jnp.where(iota == a1, NEG, logits)
    m2 = jnp.max(masked, axis=1, keepdims=True)
    a2 = jnp.min(jnp.where(masked == m2, iota, E), axis=1, keepdims=True)
    w1 = jax.nn.sigmoid(m1 - m2)
    a1_ref[...] = a1
    a2_ref[...] = a2
    w1_ref[...] = w1
    w2_ref[...] = 1.0 - w1


def _router(x2d, rwt):
    return pl.pallas_call(
        _router_body,
        out_shape=(
            jax.ShapeDtypeStruct((S, 1), jnp.int32),
            jax.ShapeDtypeStruct((S, 1), jnp.int32),
            jax.ShapeDtypeStruct((S, 1), jnp.float32),
            jax.ShapeDtypeStruct((S, 1), jnp.float32),
        ),
    )(x2d, rwt)


def _sc_gather(table, idx):
    nrows, d = table.shape
    b = idx.shape[0]
    info = plsc.get_sparse_core_info()
    nw = info.num_cores * info.num_subcores
    nc = info.num_cores
    b_per_w = b // nw
    chunk = 32
    nchunks = b_per_w // chunk

    def body(table_hbm, idx_hbm, out_hbm, idx_v, rows_v, sem):
        wid = lax.axis_index("s") * nc + lax.axis_index("c")
        base = wid * b_per_w

        def step(ci, carry):
            off = base + ci * chunk
            pltpu.sync_copy(idx_hbm.at[pl.ds(off, chunk)], idx_v)
            pltpu.async_copy(table_hbm.at[idx_v], rows_v, sem).wait()
            pltpu.sync_copy(rows_v, out_hbm.at[pl.ds(off, chunk)])
            return carry

        lax.fori_loop(0, nchunks, step, 0)

    fn = pl.kernel(
        body,
        out_type=jax.ShapeDtypeStruct((b, d), jnp.float32),
        mesh=plsc.VectorSubcoreMesh(core_axis_name="c", subcore_axis_name="s"),
        scratch_types=[
            pltpu.VMEM((chunk,), jnp.int32),
            pltpu.VMEM((chunk, d), jnp.float32),
            pltpu.SemaphoreType.DMA,
        ],
    )
    return fn(table, idx)


DFB = 128
NJ = DFF // DFB


def _gmm_body(t_ref, ex_ref, s_ref, e_ref, xs_ref, wg_ref, wu_ref, wd_ref,
              out_ref, acc_ref):
    i = pl.program_id(0)
    j = pl.program_id(1)
    tile = t_ref[i]
    prev = t_ref[jnp.maximum(i - 1, 0)]
    first_visit = jnp.logical_or(i == 0, tile != prev)
    init = jnp.logical_and(first_visit, j == 0)
    nxt = t_ref[jnp.minimum(i + 1, GSTEPS - 1)]
    flush = jnp.logical_and(j == NJ - 1,
                            jnp.logical_or(i == GSTEPS - 1, nxt != tile))
    x = xs_ref[...]
    jcol = pl.multiple_of(j * DFB, DFB)
    g = jnp.dot(x, wg_ref[0, :, pl.ds(jcol, DFB)],
                preferred_element_type=jnp.float32)
    u = jnp.dot(x, wu_ref[0], preferred_element_type=jnp.float32)
    h = jax.nn.silu(g) * u
    rows = tile * TR + lax.broadcasted_iota(jnp.int32, (TR, 1), 0)
    mask = jnp.logical_and(rows >= s_ref[i], rows < e_ref[i])
    h = jnp.where(mask, h, 0.0)
    y = jnp.dot(h, wd_ref[0], preferred_element_type=jnp.float32)

    @pl.when(init)
    def _():
        acc_ref[...] = y

    @pl.when(jnp.logical_not(init))
    def _():
        acc_ref[...] += y

    @pl.when(flush)
    def _():
        out_ref[...] = acc_ref[...]


def _gmm(step_tile, step_expert, step_s, step_e, xs, w1g, w1u, w2):
    grid_spec = pltpu.PrefetchScalarGridSpec(
        num_scalar_prefetch=4,
        grid=(GSTEPS, NJ),
        in_specs=[
            pl.BlockSpec((TR, HIDDEN), lambda i, j, t, ex, s, e: (t[i], 0)),
            pl.BlockSpec((1, HIDDEN, DFF), lambda i, j, t, ex, s, e: (ex[i], 0, 0)),
            pl.BlockSpec((1, HIDDEN, DFB), lambda i, j, t, ex, s, e: (ex[i], 0, j)),
            pl.BlockSpec((1, DFB, HIDDEN), lambda i, j, t, ex, s, e: (ex[i], j, 0)),
        ],
        out_specs=pl.BlockSpec((TR, HIDDEN), lambda i, j, t, ex, s, e: (t[i], 0)),
        scratch_shapes=[pltpu.VMEM((TR, HIDDEN), jnp.float32)],
    )
    return pl.pallas_call(
        _gmm_body,
        grid_spec=grid_spec,
        out_shape=jax.ShapeDtypeStruct((PAIRS, HIDDEN), jnp.float32),
        compiler_params=pltpu.CompilerParams(
            dimension_semantics=("arbitrary", "arbitrary"),
        ),
    )(step_tile, step_expert, step_s, step_e, xs, w1g, w1u, w2)


def _shared_body(x_ref, sg_ref, su_ref, sd_ref, out_ref):
    j = pl.program_id(1)
    x = x_ref[...]
    cdims = (((1,), (1,)), ((), ()))
    g = lax.dot_general(x, sg_ref[...], cdims, preferred_element_type=jnp.float32)
    u = lax.dot_general(x, su_ref[...], cdims, preferred_element_type=jnp.float32)
    h = jax.nn.silu(g) * u
    y = lax.dot_general(h, sd_ref[...], cdims, preferred_element_type=jnp.float32)

    @pl.when(j == 0)
    def _():
        out_ref[...] = y

    @pl.when(j != 0)
    def _():
        out_ref[...] += y


STOK = 1024


def _shared_mlp(x2d, sg, su, sd):
    return pl.pallas_call(
        _shared_body,
        grid=(S // STOK, SH // SHB),
        in_specs=[
            pl.BlockSpec((STOK, HIDDEN), lambda i, j: (i, 0)),
            pl.BlockSpec((SHB, HIDDEN), lambda i, j: (j, 0)),
            pl.BlockSpec((SHB, HIDDEN), lambda i, j: (j, 0)),
            pl.BlockSpec((HIDDEN, SHB), lambda i, j: (0, j)),
        ],
        out_specs=pl.BlockSpec((STOK, HIDDEN), lambda i, j: (i, 0)),
        out_shape=jax.ShapeDtypeStruct((S, HIDDEN), jnp.float32),
        compiler_params=pltpu.CompilerParams(
            dimension_semantics=("arbitrary", "arbitrary"),
        ),
    )(x2d, sg, su, sd)


def _combine_body(sh_ref, ga_ref, gb_ref, w0_ref, w1_ref, out_ref):
    out_ref[...] = (
        sh_ref[...] + w0_ref[...] * ga_ref[...] + w1_ref[...] * gb_ref[...]
    )


def _combine(sh, gathered, w0, w1):
    nt = S // TR
    return pl.pallas_call(
        _combine_body,
        grid=(nt,),
        in_specs=[
            pl.BlockSpec((TR, HIDDEN), lambda i: (i, 0)),
            pl.BlockSpec((TR, HIDDEN), lambda i: (i, 0)),
            pl.BlockSpec((TR, HIDDEN), lambda i, _nt=nt: (i + _nt, 0)),
            pl.BlockSpec((TR, 1), lambda i: (i, 0)),
            pl.BlockSpec((TR, 1), lambda i: (i, 0)),
        ],
        out_specs=pl.BlockSpec((TR, HIDDEN), lambda i: (i, 0)),
        out_shape=jax.ShapeDtypeStruct((S, HIDDEN), jnp.float32),
    )(sh, gathered, gathered, w0, w1)


def _route_metadata(a1, a2):
    flat_e = jnp.stack([a1[:, 0], a2[:, 0]], axis=1).reshape(-1)
    order = jnp.argsort(flat_e, stable=True)
    sorted_tokens = (order // 2).astype(jnp.int32)
    inv = jnp.argsort(order).astype(jnp.int32)
    inv2 = inv.reshape(S, 2)
    comb_idx = jnp.concatenate([inv2[:, 0], inv2[:, 1]])

    counts = jnp.bincount(flat_e, length=E)
    ends = jnp.cumsum(counts)
    starts = ends - counts
    ntiles = jnp.where(counts > 0, (ends - 1) // TR - starts // TR + 1, 0)
    cum = jnp.cumsum(ntiles)
    i_arr = jnp.arange(GSTEPS)
    e_i = jnp.searchsorted(cum, i_arr, side="right")
    valid = i_arr < cum[E - 1]
    e_c = jnp.minimum(e_i, E - 1)
    prevcum = jnp.where(e_c > 0, cum[jnp.maximum(e_c - 1, 0)], 0)
    tile_i = starts[e_c] // TR + (i_arr - prevcum)
    step_tile = jnp.where(valid, tile_i, NTILES - 1).astype(jnp.int32)
    step_expert = jnp.where(valid, e_c, 0).astype(jnp.int32)
    step_s = jnp.where(valid, starts[e_c], 0).astype(jnp.int32)
    step_e = jnp.where(valid, ends[e_c], 0).astype(jnp.int32)
    return sorted_tokens, comb_idx, step_tile, step_expert, step_s, step_e


def kernel(hidden_states, router_weight, w1_gate, w1_up, w2,
           shared_gate, shared_up, shared_down):
    orig_shape = hidden_states.shape
    x2d = hidden_states.reshape(-1, HIDDEN)

    a1, a2, w1c, w2c = _router(x2d, router_weight.T)
    (sorted_tokens, comb_idx, step_tile, step_expert,
     step_s, step_e) = _route_metadata(a1, a2)

    xs = _sc_gather(x2d, sorted_tokens)
    ys = _gmm(step_tile, step_expert, step_s, step_e, xs, w1_gate, w1_up, w2)
    gathered = _sc_gather(ys, comb_idx)
    sh = _shared_mlp(x2d, shared_gate, shared_up, shared_down)
    out = _combine(sh, gathered, w1c, w2c)
    return out.reshape(orig_shape)


# speedup vs baseline: 1.5159x; 1.5159x over previous
import functools

import jax
import jax.numpy as jnp
from jax import lax
from jax.experimental import pallas as pl
from jax.experimental.pallas import tpu as pltpu
from jax.experimental.pallas import tpu_sc as plsc

HIDDEN = 2048
E = 16
TOPK = 2
DFF = 1408
SH = DFF * 2
S = 2048
PAIRS = S * TOPK
TR = 256
NTILES = PAIRS // TR
GSTEPS = NTILES + E - 1
SHB = 256
NEG = -1e30


def _router_body(x_ref, rwt_ref, a1_ref, a2_ref, w1_ref, w2_ref):
    logits = jnp.dot(x_ref[...], rwt_ref[...], preferred_element_type=jnp.float32)
    iota = lax.broadcasted_iota(jnp.int32, (S, E), 1)
    m1 = jnp.max(logits, axis=1, keepdims=True)
    a1 = jnp.min(jnp.where(logits == m1, iota, E), axis=1, keepdims=True)
    masked = jnp.where(iota == a1, NEG, logits)
    m2 = jnp.max(masked, axis=1, keepdims=True)
    a2 = jnp.min(jnp.where(masked == m2, iota, E), axis=1, keepdims=True)
    w1 = jax.nn.sigmoid(m1 - m2)
    a1_ref[...] = a1
    a2_ref[...] = a2
    w1_ref[...] = w1
    w2_ref[...] = 1.0 - w1


def _router(x2d, rwt):
    return pl.pallas_call(
        _router_body,
        out_shape=(
            jax.ShapeDtypeStruct((S, 1), jnp.int32),
            jax.ShapeDtypeStruct((S, 1), jnp.int32),
            jax.ShapeDtypeStruct((S, 1), jnp.float32),
            jax.ShapeDtypeStruct((S, 1), jnp.float32),
        ),
    )(x2d, rwt)


def _sc_gather(table, idx):
    nrows, d = table.shape
    b = idx.shape[0]
    info = plsc.get_sparse_core_info()
    nw = info.num_cores * info.num_subcores
    nc = info.num_cores
    b_per_w = b // nw
    chunk = 32
    nchunks = b_per_w // chunk

    def body(table_hbm, idx_hbm, out_hbm, idx_v, rows_v, sem):
        wid = lax.axis_index("s") * nc + lax.axis_index("c")
        base = wid * b_per_w

        def step(ci, carry):
            off = base + ci * chunk
            pltpu.sync_copy(idx_hbm.at[pl.ds(off, chunk)], idx_v)
            pltpu.async_copy(table_hbm.at[idx_v], rows_v, sem).wait()
            pltpu.sync_copy(rows_v, out_hbm.at[pl.ds(off, chunk)])
            return carry

        lax.fori_loop(0, nchunks, step, 0)

    fn = pl.kernel(
        body,
        out_type=jax.ShapeDtypeStruct((b, d), jnp.float32),
        mesh=plsc.VectorSubcoreMesh(core_axis_name="c", subcore_axis_name="s"),
        scratch_types=[
            pltpu.VMEM((chunk,), jnp.int32),
            pltpu.VMEM((chunk, d), jnp.float32),
            pltpu.SemaphoreType.DMA,
        ],
    )
    return fn(table, idx)


def _gmm_a_body(t_ref, ex_ref, s_ref, e_ref, xs_ref, wg_ref, wu_ref, h_ref):
    i = pl.program_id(0)
    tile = t_ref[i]
    prev = t_ref[jnp.maximum(i - 1, 0)]
    first_visit = jnp.logical_or(i == 0, tile != prev)
    x = xs_ref[...]
    g = jnp.dot(x, wg_ref[0], preferred_element_type=jnp.float32)
    u = jnp.dot(x, wu_ref[0], preferred_element_type=jnp.float32)
    h = jax.nn.silu(g) * u
    rows = tile * TR + lax.broadcasted_iota(jnp.int32, (TR, 1), 0)
    mask = jnp.logical_and(rows >= s_ref[i], rows < e_ref[i])
    h = jnp.where(mask, h, 0.0)

    @pl.when(first_visit)
    def _():
        h_ref[...] = h

    @pl.when(jnp.logical_not(first_visit))
    def _():
        h_ref[...] += h


def _gmm_b_body(t_ref, ex_ref, s_ref, e_ref, h_ref, wd_ref, out_ref, acc_ref):
    i = pl.program_id(0)
    tile = t_ref[i]
    prev = t_ref[jnp.maximum(i - 1, 0)]
    first_visit = jnp.logical_or(i == 0, tile != prev)
    nxt = t_ref[jnp.minimum(i + 1, GSTEPS - 1)]
    flush = jnp.logical_or(i == GSTEPS - 1, nxt != tile)
    y = jnp.dot(h_ref[...], wd_ref[0], preferred_element_type=jnp.float32)
    rows = tile * TR + lax.broadcasted_iota(jnp.int32, (TR, 1), 0)
    mask = jnp.logical_and(rows >= s_ref[i], rows < e_ref[i])
    y = jnp.where(mask, y, 0.0)

    @pl.when(first_visit)
    def _():
        acc_ref[...] = y

    @pl.when(jnp.logical_not(first_visit))
    def _():
        acc_ref[...] += y

    @pl.when(flush)
    def _():
        out_ref[...] = acc_ref[...]


def _gmm(step_tile, step_expert, step_s, step_e, xs, w1g, w1u, w2):
    spec_a = pltpu.PrefetchScalarGridSpec(
        num_scalar_prefetch=4,
        grid=(GSTEPS,),
        in_specs=[
            pl.BlockSpec((TR, HIDDEN), lambda i, t, ex, s, e: (t[i], 0)),
            pl.BlockSpec((1, HIDDEN, DFF), lambda i, t, ex, s, e: (ex[i], 0, 0)),
            pl.BlockSpec((1, HIDDEN, DFF), lambda i, t, ex, s, e: (ex[i], 0, 0)),
        ],
        out_specs=pl.BlockSpec((TR, DFF), lambda i, t, ex, s, e: (t[i], 0)),
    )
    hbuf = pl.pallas_call(
        _gmm_a_body,
        grid_spec=spec_a,
        out_shape=jax.ShapeDtypeStruct((PAIRS, DFF), jnp.float32),
        compiler_params=pltpu.CompilerParams(
            dimension_semantics=("arbitrary",),
        ),
    )(step_tile, step_expert, step_s, step_e, xs, w1g, w1u)

    spec_b = pltpu.PrefetchScalarGridSpec(
        num_scalar_prefetch=4,
        grid=(GSTEPS,),
        in_specs=[
            pl.BlockSpec((TR, DFF), lambda i, t, ex, s, e: (t[i], 0)),
            pl.BlockSpec((1, DFF, HIDDEN), lambda i, t, ex, s, e: (ex[i], 0, 0)),
        ],
        out_specs=pl.BlockSpec((TR, HIDDEN), lambda i, t, ex, s, e: (t[i], 0)),
        scratch_shapes=[pltpu.VMEM((TR, HIDDEN), jnp.float32)],
    )
    return pl.pallas_call(
        _gmm_b_body,
        grid_spec=spec_b,
        out_shape=jax.ShapeDtypeStruct((PAIRS, HIDDEN), jnp.float32),
        compiler_params=pltpu.CompilerParams(
            dimension_semantics=("arbitrary",),
        ),
    )(step_tile, step_expert, step_s, step_e, hbuf, w2)


def _shared_body(x_ref, sg_ref, su_ref, sd_ref, out_ref):
    j = pl.program_id(1)
    x = x_ref[...]
    cdims = (((1,), (1,)), ((), ()))
    g = lax.dot_general(x, sg_ref[...], cdims, preferred_element_type=jnp.float32)
    u = lax.dot_general(x, su_ref[...], cdims, preferred_element_type=jnp.float32)
    h = jax.nn.silu(g) * u
    y = lax.dot_general(h, sd_ref[...], cdims, preferred_element_type=jnp.float32)

    @pl.when(j == 0)
    def _():
        out_ref[...] = y

    @pl.when(j != 0)
    def _():
        out_ref[...] += y


STOK = 1024


def _shared_mlp(x2d, sg, su, sd):
    return pl.pallas_call(
        _shared_body,
        grid=(S // STOK, SH // SHB),
        in_specs=[
            pl.BlockSpec((STOK, HIDDEN), lambda i, j: (i, 0)),
            pl.BlockSpec((SHB, HIDDEN), lambda i, j: (j, 0)),
            pl.BlockSpec((SHB, HIDDEN), lambda i, j: (j, 0)),
            pl.BlockSpec((HIDDEN, SHB), lambda i, j: (0, j)),
        ],
        out_specs=pl.BlockSpec((STOK, HIDDEN), lambda i, j: (i, 0)),
        out_shape=jax.ShapeDtypeStruct((S, HIDDEN), jnp.float32),
        compiler_params=pltpu.CompilerParams(
            dimension_semantics=("arbitrary", "arbitrary"),
        ),
    )(x2d, sg, su, sd)


def _combine_body(sh_ref, ga_ref, gb_ref, w0_ref, w1_ref, out_ref):
    out_ref[...] = (
        sh_ref[...] + w0_ref[...] * ga_ref[...] + w1_ref[...] * gb_ref[...]
    )


def _combine(sh, gathered, w0, w1):
    nt = S // TR
    return pl.pallas_call(
        _combine_body,
        grid=(nt,),
        in_specs=[
            pl.BlockSpec((TR, HIDDEN), lambda i: (i, 0)),
            pl.BlockSpec((TR, HIDDEN), lambda i: (i, 0)),
            pl.BlockSpec((TR, HIDDEN), lambda i, _nt=nt: (i + _nt, 0)),
            pl.BlockSpec((TR, 1), lambda i: (i, 0)),
            pl.BlockSpec((TR, 1), lambda i: (i, 0)),
        ],
        out_specs=pl.BlockSpec((TR, HIDDEN), lambda i: (i, 0)),
        out_shape=jax.ShapeDtypeStruct((S, HIDDEN), jnp.float32),
    )(sh, gathered, gathered, w0, w1)


def _route_metadata(a1, a2):
    flat_e = jnp.stack([a1[:, 0], a2[:, 0]], axis=1).reshape(-1)
    order = jnp.argsort(flat_e, stable=True)
    sorted_tokens = (order // 2).astype(jnp.int32)
    inv = jnp.argsort(order).astype(jnp.int32)
    inv2 = inv.reshape(S, 2)
    comb_idx = jnp.concatenate([inv2[:, 0], inv2[:, 1]])

    counts = jnp.bincount(flat_e, length=E)
    ends = jnp.cumsum(counts)
    starts = ends - counts
    ntiles = jnp.where(counts > 0, (ends - 1) // TR - starts // TR + 1, 0)
    cum = jnp.cumsum(ntiles)
    i_arr = jnp.arange(GSTEPS)
    e_i = jnp.searchsorted(cum, i_arr, side="right")
    valid = i_arr < cum[E - 1]
    e_c = jnp.minimum(e_i, E - 1)
    prevcum = jnp.where(e_c > 0, cum[jnp.maximum(e_c - 1, 0)], 0)
    tile_i = starts[e_c] // TR + (i_arr - prevcum)
    step_tile = jnp.where(valid, tile_i, NTILES - 1).astype(jnp.int32)
    step_expert = jnp.where(valid, e_c, 0).astype(jnp.int32)
    step_s = jnp.where(valid, starts[e_c], 0).astype(jnp.int32)
    step_e = jnp.where(valid, ends[e_c], 0).astype(jnp.int32)
    return sorted_tokens, comb_idx, step_tile, step_expert, step_s, step_e


def kernel(hidden_states, router_weight, w1_gate, w1_up, w2,
           shared_gate, shared_up, shared_down):
    orig_shape = hidden_states.shape
    x2d = hidden_states.reshape(-1, HIDDEN)

    a1, a2, w1c, w2c = _router(x2d, router_weight.T)
    (sorted_tokens, comb_idx, step_tile, step_expert,
     step_s, step_e) = _route_metadata(a1, a2)

    xs = _sc_gather(x2d, sorted_tokens)
    ys = _gmm(step_tile, step_expert, step_s, step_e, xs, w1_gate, w1_up, w2)
    gathered = _sc_gather(ys, comb_idx)
    sh = _shared_mlp(x2d, shared_gate, shared_up, shared_down)
    out = _combine(sh, gathered, w1c, w2c)
    return out.reshape(orig_shape)
